# baseline (speedup 1.0000x reference)
import jax
import jax.numpy as jnp
from jax.experimental import pallas as pl

N_ENT = 100000
RANK = 128
NNZ = 500000
SHAPE = (N_ENT, N_ENT, N_ENT)


def _loss_deriv_body(m_ref, x_ref, loss_ref, deriv_ref):
    m = m_ref[...]
    x = x_ref[...]
    loss_ref[...] = jnp.logaddexp(0.0, m) - x * m
    deriv_ref[...] = jax.nn.sigmoid(m) - x


def _loss_deriv(m, x):
    n = m.shape[0]
    npad = (n + 127) // 128 * 128
    m2 = jnp.pad(m, (0, npad - n)).reshape(npad // 128, 128)
    x2 = jnp.pad(x, (0, npad - n)).reshape(npad // 128, 128)
    loss2, deriv2 = pl.pallas_call(
        _loss_deriv_body,
        out_shape=(
            jax.ShapeDtypeStruct(m2.shape, jnp.float32),
            jax.ShapeDtypeStruct(m2.shape, jnp.float32),
        ),
    )(m2, x2)
    return loss2.reshape(-1)[:n], deriv2.reshape(-1)[:n]


def _mttcrp(coo, vals, mode, f1, f2):
    if mode == 0:
        m1, m2 = 1, 2
    elif mode == 1:
        m1, m2 = 0, 2
    else:
        m1, m2 = 0, 1
    contrib = vals[:, None] * f1[coo[:, m1], :] * f2[coo[:, m2], :]
    acc = jnp.zeros((SHAPE[mode], f1.shape[1]), dtype=contrib.dtype)
    acc = acc.at[coo[:, mode]].add(contrib)
    return acc[coo[:, mode], :]


def kernel(coo_ns, vals_ns, a_elems, b_elems, c_elems, a, b):
    kruskal_val = jnp.sum(
        a[coo_ns[:, 0], :] * b[coo_ns[:, 1], :] * a[coo_ns[:, 2], :], axis=1
    )
    loss, deriv = _loss_deriv(kruskal_val, vals_ns)
    g_a = _mttcrp(coo_ns, deriv, 0, b, a)
    g_b = _mttcrp(coo_ns, deriv, 1, a, a)
    g_c = _mttcrp(coo_ns, deriv, 2, a, b)
    a_grad = jnp.zeros_like(a).at[a_elems, :].set(g_a).at[c_elems, :].set(g_c)
    b_grad = jnp.zeros_like(b).at[b_elems, :].set(g_b)
    return (loss, a_grad, b_grad)


# speedup vs baseline: 1.3882x; 1.3882x over previous
import functools

import jax
import jax.numpy as jnp
from jax import lax
from jax.experimental import pallas as pl
from jax.experimental.pallas import tpu as pltpu
from jax.experimental.pallas import tpu_sc as plsc

N_ENT = 100000
RANK = 128
NNZ = 500000

NC = 2
NS = 16
NW = NC * NS
L = 16

NNZ_PAD = 512000
CHUNK1 = NNZ_PAD // NW
B1 = 64
GB = 10
NGRP = CHUNK1 // (B1 * GB)

SENT_DEST = 1 << 20

TBL = 102400
SEG = TBL // NW

R2 = 6656
NP2 = 16
ACCM = R2 * NP2
ZROW = 3 * ACCM
ACC_ROWS = 3 * ACCM + L
R2T = (R2 + L) // NS
R2W = R2 // NS
SCAN2 = NNZ_PAD // NS
SEC2 = 8000
HMAX = SEC2 + 2 * 128

_mesh = functools.partial(
    plsc.VectorSubcoreMesh, core_axis_name="c", subcore_axis_name="s",
    num_cores=NC, num_subcores=NS)
_SC_PARAMS = pltpu.CompilerParams(needs_layout_passes=False)


def _wid():
    return lax.axis_index("s") * NC + lax.axis_index("c")


def _iota():
    return lax.iota(jnp.int32, L)


def _sc1_body(colsg, valsp, a_hbm, b_hbm, m_out, co_out,
              idxb, valb, rows, cob, mb, db, mrow, gsem, wsem):
    wid = _wid()
    wbase = wid * CHUNK1
    srcs = (a_hbm, b_hbm, a_hbm)

    def group(g, _):
        gbase = wbase + g * (B1 * GB)
        for t in range(3):
            pltpu.sync_copy(
                colsg.at[pl.ds(t * NNZ_PAD + gbase, B1 * GB)],
                idxb.at[pl.ds(t * B1 * GB, B1 * GB)])
        pltpu.sync_copy(valsp.at[pl.ds(gbase, B1 * GB)], valb)

        def issue_gather(bb):
            s = bb % 2
            descs = []
            for t in range(3):
                descs.append(pltpu.async_copy(
                    srcs[t].at[idxb.at[pl.ds(t * B1 * GB + bb * B1, B1)]],
                    rows.at[s, t], gsem))
            return descs

        gdesc = {0: issue_gather(0)}
        wdesc = {}
        for bb in range(GB):
            s = bb % 2
            if bb + 1 < GB:
                gdesc[bb + 1] = issue_gather(bb + 1)
            for d in gdesc.pop(bb):
                d.wait()
            if bb - 2 in wdesc:
                for d in wdesc.pop(bb - 2):
                    d.wait()

            iot = _iota()

            def q_body(q, _):
                jb = q * 16

                def p1_body(j, _):
                    jj = jb + j
                    macc = jnp.zeros((L,), jnp.float32)
                    for k in range(8):
                        va0 = rows[s, 0, jj, pl.ds(k * L, L)]
                        vb1 = rows[s, 1, jj, pl.ds(k * L, L)]
                        va2 = rows[s, 2, jj, pl.ds(k * L, L)]
                        macc = macc + va0 * (vb1 * va2)
                    mrow[pl.ds(j * L, L)] = macc
                    return 0

                lax.fori_loop(0, 16, p1_body, 0, unroll=False)
                mv = jnp.zeros((L,), jnp.float32)
                for k in range(L):
                    mv = mv + plsc.load_gather(mrow, [iot * L + k])
                mb[pl.ds(s * B1 + jb, L)] = mv
                xv = valb[pl.ds(bb * B1 + jb, L)]
                dv = 1.0 / (1.0 + jnp.exp(-mv)) - xv
                db[...] = dv

                def p2_body(j, _):
                    jj = jb + j
                    dvb = plsc.load_gather(db, [jnp.full((L,), j, jnp.int32)])
                    for k in range(8):
                        kd = pl.ds(k * L, L)
                        va0 = rows[s, 0, jj, kd]
                        vb1 = rows[s, 1, jj, kd]
                        va2 = rows[s, 2, jj, kd]
                        cob[s, 0, jj, kd] = dvb * (vb1 * va2)
                        cob[s, 1, jj, kd] = dvb * (va0 * va2)
                        cob[s, 2, jj, kd] = dvb * (va0 * vb1)
                    return 0

                lax.fori_loop(0, 16, p2_body, 0, unroll=False)
                return 0

            lax.fori_loop(0, 4, q_body, 0, unroll=False)

            blk = gbase + bb * B1
            wd = []
            for t in range(3):
                wd.append(pltpu.async_copy(
                    cob.at[s, t], co_out.at[pl.ds(t * NNZ_PAD + blk, B1)],
                    wsem))
            wd.append(pltpu.async_copy(
                mb.at[pl.ds(s * B1, B1)], m_out.at[pl.ds(blk, B1)], wsem))
            wdesc[bb] = wd

        for bb in sorted(wdesc):
            for d in wdesc[bb]:
                d.wait()
        return 0

    lax.fori_loop(0, NGRP, group, 0, unroll=False)


def _sc1(colsg, valsp, a, b):
    f = pl.kernel(
        _sc1_body,
        out_type=(
            jax.ShapeDtypeStruct((NNZ_PAD,), jnp.float32),
            jax.ShapeDtypeStruct((3 * NNZ_PAD, RANK), jnp.float32),
        ),
        mesh=_mesh(),
        compiler_params=_SC_PARAMS,
        scratch_types=[
            pltpu.VMEM((3 * B1 * GB,), jnp.int32),
            pltpu.VMEM((B1 * GB,), jnp.float32),
            pltpu.VMEM((2, 3, B1, RANK), jnp.float32),
            pltpu.VMEM((2, 3, B1, RANK), jnp.float32),
            pltpu.VMEM((2 * B1,), jnp.float32),
            pltpu.VMEM((L,), jnp.float32),
            pltpu.VMEM((L * L,), jnp.float32),
            pltpu.SemaphoreType.DMA,
            pltpu.SemaphoreType.DMA,
        ],
    )
    return f(colsg, valsp, a, b)


def _tcl_body(m_ref, x_ref, loss_ref):
    m = m_ref[...]
    x = x_ref[...]
    loss_ref[...] = jnp.logaddexp(0.0, m) - x * m


def _tc_loss(m_pad, x_pad):
    m2 = m_pad.reshape(NNZ_PAD // RANK, RANK)
    x2 = x_pad.reshape(NNZ_PAD // RANK, RANK)
    loss2 = pl.pallas_call(
        _tcl_body,
        out_shape=jax.ShapeDtypeStruct(m2.shape, jnp.float32),
    )(m2, x2)
    return loss2.reshape(-1)[:NNZ]


def _w1_body(elems3, w1_out, tbl, ebuf, esb):
    wid = _wid()
    wbase = wid * CHUNK1
    iot = _iota()

    for arr in range(3):
        def init(v, _):
            tbl[pl.ds(v * L, L)] = jnp.full((L,), -1, jnp.int32)
            return 0

        lax.fori_loop(0, TBL // L, init, 0, unroll=False)

        def ck_body(ck, _):
            base = wbase + ck * 2000
            pltpu.sync_copy(elems3.at[pl.ds(arr * NNZ_PAD + base, 2000)], ebuf)

            def v_body(v, _):
                e = ebuf[pl.ds(v * L, L)]
                key = (e << 4) | iot
                val = (base + v * L) + iot
                ks, vs = plsc.sort_key_val(key, val)
                es = ks >> 4
                esb[pl.ds(0, L)] = es
                esb[pl.ds(L, L)] = jnp.full((L,), 1 << 30, jnp.int32)
                nxt = esb[pl.ds(1, L)]
                keep = es != nxt
                plsc.store_scatter(tbl, [es], vs, mask=keep)
                return 0

            lax.fori_loop(0, 2000 // L, v_body, 0, unroll=False)
            return 0

        lax.fori_loop(0, CHUNK1 // 2000, ck_body, 0, unroll=False)
        pltpu.sync_copy(tbl, w1_out.at[pl.ds((arr * NW + wid) * TBL, TBL)])


def _w1(elems3):
    f = pl.kernel(
        _w1_body,
        out_type=jax.ShapeDtypeStruct((3 * NW * TBL,), jnp.int32),
        mesh=_mesh(),
        compiler_params=_SC_PARAMS,
        scratch_types=[
            pltpu.VMEM((TBL,), jnp.int32),
            pltpu.VMEM((2000,), jnp.int32),
            pltpu.VMEM((2 * L,), jnp.int32),
        ],
    )
    return f(elems3)


def _sc2_body(colsd, contrib, acc_out,
              chunk, zb, destb, ilist, dlist, drow, grows):
    core = lax.axis_index("c")
    sid = lax.axis_index("s")
    iot = _iota()

    def z_row(v, _):
        def z_col(k, _):
            zb[v, pl.ds(k * L, L)] = jnp.zeros((L,), jnp.float32)
            return 0
        lax.fori_loop(0, RANK // L, z_col, 0, unroll=False)
        return 0

    lax.fori_loop(0, 64, z_row, 0, unroll=False)

    def pair_body(it, _):
        pair = it * NC + core
        m = pair // NP2
        p = pair % NP2
        rowbase = p * R2

        zbase = sid * R2T
        for z in range(6):
            pltpu.sync_copy(zb.at[pl.ds(0, 64)], chunk.at[pl.ds(zbase + z * 64, 64)])
        pltpu.sync_copy(zb.at[pl.ds(0, R2T - 6 * 64)],
                        chunk.at[pl.ds(zbase + 6 * 64, R2T - 6 * 64)])
        plsc.subcore_barrier()

        def sec_body(sec, _):
            sbase = m * NNZ_PAD + sid * SCAN2 + sec * SEC2

            def ck_body(ck, nh):
                pltpu.sync_copy(colsd.at[pl.ds(sbase + ck * 2000, 2000)], destb)

                def v_body(v, nh):
                    dv = destb[pl.ds(v * L, L)]
                    rel = dv - rowbase
                    msk = (rel >= 0) & (rel < R2)
                    iv = (sbase + ck * 2000 + v * L) + iot
                    plsc.store_compressed(ilist.at[pl.ds(nh, L)], iv, mask=msk)
                    plsc.store_compressed(dlist.at[pl.ds(nh, L)], rel, mask=msk)
                    cnt = plsc.all_reduce_population_count(msk)
                    return nh + cnt[0]

                return lax.fori_loop(0, 2000 // L, v_body, nh, unroll=False)

            nhits = lax.fori_loop(0, SEC2 // 2000, ck_body, 0, unroll=False)

            def pad_body(k, _):
                dum = jnp.full((L,), R2 + sid, jnp.int32)
                zi = jnp.zeros((L,), jnp.int32)
                ilist[pl.ds(nhits + k * L, L)] = zi
                dlist[pl.ds(nhits + k * L, L)] = dum
                return 0

            lax.fori_loop(0, 8, pad_body, 0, unroll=False)
            nbat = (nhits + 127) // 128

            def b_body(bi, _):
                def cp(k, _):
                    drow[0, pl.ds(k * L, L)] = dlist[pl.ds(bi * 128 + k * L, L)]
                    return 0
                lax.fori_loop(0, 128 // L, cp, 0, unroll=False)
                pltpu.sync_copy(contrib.at[ilist.at[pl.ds(bi * 128, 128)]], grows)
                pltpu.sync_copy(grows, chunk.at[drow.at[0]], add=True)
                return 0

            lax.fori_loop(0, nbat, b_body, 0, unroll=False)
            return 0

        lax.fori_loop(0, SCAN2 // SEC2, sec_body, 0, unroll=False)
        plsc.subcore_barrier()

        pltpu.sync_copy(
            chunk.at[pl.ds(sid * R2W, R2W)],
            acc_out.at[pl.ds(m * ACCM + p * R2 + sid * R2W, R2W)])
        plsc.subcore_barrier()
        return 0

    lax.fori_loop(0, (3 * NP2) // NC, pair_body, 0, unroll=False)

    @pl.when((core == 0) & (sid == 0))
    def _():
        pltpu.sync_copy(zb.at[pl.ds(0, L)], acc_out.at[pl.ds(ZROW, L)])


def _sc2(colsd, contrib):
    f = pl.kernel(
        _sc2_body,
        out_type=jax.ShapeDtypeStruct((ACC_ROWS, RANK), jnp.float32),
        mesh=_mesh(),
        compiler_params=_SC_PARAMS,
        scratch_types=[
            pltpu.VMEM_SHARED((R2 + L, RANK), jnp.float32),
            pltpu.VMEM((64, RANK), jnp.float32),
            pltpu.VMEM((2000,), jnp.int32),
            pltpu.VMEM((HMAX,), jnp.int32),
            pltpu.VMEM((HMAX,), jnp.int32),
            pltpu.VMEM((1, 128), jnp.int32),
            pltpu.VMEM((128, RANK), jnp.float32),
        ],
    )
    return f(colsd, contrib)


def _sc3_body(w1, colsg, acc, a_out, b_out,
              winb, seg, cvb, srca, srcb, widx, grows):
    wid = _wid()
    ebase = wid * SEG

    for arr in range(3):
        def init(v, _):
            winb[pl.ds(arr * SEG + v * L, L)] = jnp.full((L,), -1, jnp.int32)
            return 0
        lax.fori_loop(0, SEG // L, init, 0, unroll=False)

        def t_body(t, _):
            pltpu.sync_copy(w1.at[pl.ds((arr * NW + t) * TBL + ebase, SEG)], seg)

            def v_body(v, _):
                wd = pl.ds(arr * SEG + v * L, L)
                vd = pl.ds(v * L, L)
                winb[wd] = jnp.maximum(winb[wd], seg[vd])
                return 0
            lax.fori_loop(0, SEG // L, v_body, 0, unroll=False)
            return 0
        lax.fori_loop(0, NW, t_body, 0, unroll=False)

    for arr in range(3):
        def c_body(v, _):
            wd = pl.ds(arr * SEG + v * L, L)
            widx[wd] = jnp.maximum(winb[wd], 0) + arr * NNZ_PAD
            return 0
        lax.fori_loop(0, SEG // L, c_body, 0, unroll=False)

        def g_body(bi, _):
            pltpu.sync_copy(
                colsg.at[widx.at[pl.ds(arr * SEG + bi * 128, 128)]],
                cvb.at[pl.ds(arr * SEG + bi * 128, 128)])
            return 0
        lax.fori_loop(0, SEG // 128, g_body, 0, unroll=False)

    def s_body(v, _):
        vd = pl.ds(v * L, L)
        wa = winb[pl.ds(0 * SEG + v * L, L)]
        wb = winb[pl.ds(1 * SEG + v * L, L)]
        wc = winb[pl.ds(2 * SEG + v * L, L)]
        cv0 = cvb[pl.ds(0 * SEG + v * L, L)]
        cv1 = cvb[pl.ds(1 * SEG + v * L, L)]
        cv2 = cvb[pl.ds(2 * SEG + v * L, L)]
        zr = jnp.full((L,), ZROW, jnp.int32)
        sa = jnp.where(wa >= 0, cv0, zr)
        sa = jnp.where(wc >= 0, 2 * ACCM + cv2, sa)
        sb = jnp.where(wb >= 0, ACCM + cv1, zr)
        srca[vd] = sa
        srcb[vd] = sb
        return 0

    lax.fori_loop(0, SEG // L, s_body, 0, unroll=False)

    def f_body(bi, _):
        pltpu.sync_copy(acc.at[srca.at[pl.ds(bi * 128, 128)]], grows)
        pltpu.sync_copy(grows, a_out.at[pl.ds(ebase + bi * 128, 128)])
        pltpu.sync_copy(acc.at[srcb.at[pl.ds(bi * 128, 128)]], grows)
        pltpu.sync_copy(grows, b_out.at[pl.ds(ebase + bi * 128, 128)])
        return 0

    lax.fori_loop(0, SEG // 128, f_body, 0, unroll=False)


def _sc3(w1, colsg, acc):
    f = pl.kernel(
        _sc3_body,
        out_type=(
            jax.ShapeDtypeStruct((TBL, RANK), jnp.float32),
            jax.ShapeDtypeStruct((TBL, RANK), jnp.float32),
        ),
        mesh=_mesh(),
        compiler_params=_SC_PARAMS,
        scratch_types=[
            pltpu.VMEM((3 * SEG,), jnp.int32),
            pltpu.VMEM((SEG,), jnp.int32),
            pltpu.VMEM((3 * SEG,), jnp.int32),
            pltpu.VMEM((SEG,), jnp.int32),
            pltpu.VMEM((SEG,), jnp.int32),
            pltpu.VMEM((3 * SEG,), jnp.int32),
            pltpu.VMEM((128, RANK), jnp.float32),
        ],
    )
    return f(w1, colsg, acc)


def kernel(coo_ns, vals_ns, a_elems, b_elems, c_elems, a, b):
    pad = NNZ_PAD - NNZ
    c0 = coo_ns[:, 0]
    c1 = coo_ns[:, 1]
    c2 = coo_ns[:, 2]
    colsg = jnp.concatenate([
        jnp.pad(c0, (0, pad)), jnp.pad(c1, (0, pad)), jnp.pad(c2, (0, pad))])
    colsd = jnp.concatenate([
        jnp.pad(c0, (0, pad), constant_values=SENT_DEST),
        jnp.pad(c1, (0, pad), constant_values=SENT_DEST),
        jnp.pad(c2, (0, pad), constant_values=SENT_DEST)])
    elems3 = jnp.concatenate([
        jnp.pad(a_elems, (0, pad), constant_values=N_ENT),
        jnp.pad(b_elems, (0, pad), constant_values=N_ENT),
        jnp.pad(c_elems, (0, pad), constant_values=N_ENT)])
    valsp = jnp.pad(vals_ns, (0, pad))

    m_pad, contrib = _sc1(colsg, valsp, a, b)
    loss = _tc_loss(m_pad, valsp)
    w1 = _w1(elems3)
    acc = _sc2(colsd, contrib)
    a_grad_pad, b_grad_pad = _sc3(w1, colsg, acc)
    return (loss, a_grad_pad[:N_ENT], b_grad_pad[:N_ENT])


# speedup vs baseline: 3.5770x; 2.5767x over previous
import functools

import jax
import jax.numpy as jnp
from jax import lax
from jax.experimental import pallas as pl
from jax.experimental.pallas import tpu as pltpu
from jax.experimental.pallas import tpu_sc as plsc

N_ENT = 100000
RANK = 128
NNZ = 500000

NC = 2
NS = 16
NW = NC * NS
L = 16

NNZ_PAD = 512000
CHUNK1 = NNZ_PAD // NW
B1 = 64
GB = 10
NGRP = CHUNK1 // (B1 * GB)

SENT_DEST = 1 << 20

TBL = 102400
SEG = TBL // NW

R2 = 6656
NP2 = 16
ACCM = R2 * NP2
ZROW = 3 * ACCM
ACC_ROWS = 3 * ACCM + L
R2T = (R2 + L) // NS
R2W = R2 // NS
SCAN2 = NNZ_PAD // NS
KSB = 2
HMAX = SCAN2 + KSB * 128
RBITS = 13

_mesh = functools.partial(
    plsc.VectorSubcoreMesh, core_axis_name="c", subcore_axis_name="s",
    num_cores=NC, num_subcores=NS)
_SC_PARAMS = pltpu.CompilerParams(needs_layout_passes=False)


def _wid():
    return lax.axis_index("s") * NC + lax.axis_index("c")


def _iota():
    return lax.iota(jnp.int32, L)


def _sc1_body(colsg, valsp, a_hbm, b_hbm, m_out, co_out,
              idxb, valb, rows, cob, mb, db, mrow, gsem, wsem):
    wid = _wid()
    wbase = wid * CHUNK1
    srcs = (a_hbm, b_hbm, a_hbm)

    def group(g, _):
        gbase = wbase + g * (B1 * GB)
        for t in range(3):
            pltpu.sync_copy(
                colsg.at[pl.ds(t * NNZ_PAD + gbase, B1 * GB)],
                idxb.at[pl.ds(t * B1 * GB, B1 * GB)])
        pltpu.sync_copy(valsp.at[pl.ds(gbase, B1 * GB)], valb)

        def issue_gather(bb):
            s = bb % 2
            descs = []
            for t in range(3):
                descs.append(pltpu.async_copy(
                    srcs[t].at[idxb.at[pl.ds(t * B1 * GB + bb * B1, B1)]],
                    rows.at[s, t], gsem))
            return descs

        gdesc = {0: issue_gather(0)}
        wdesc = {}
        for bb in range(GB):
            s = bb % 2
            if bb + 1 < GB:
                gdesc[bb + 1] = issue_gather(bb + 1)
            for d in gdesc.pop(bb):
                d.wait()
            if bb - 2 in wdesc:
                for d in wdesc.pop(bb - 2):
                    d.wait()

            iot = _iota()

            def q_body(q, _):
                jb = q * 16

                def p1_body(j, _):
                    jj = jb + j
                    macc = jnp.zeros((L,), jnp.float32)
                    for k in range(8):
                        va0 = rows[s, 0, jj, pl.ds(k * L, L)]
                        vb1 = rows[s, 1, jj, pl.ds(k * L, L)]
                        va2 = rows[s, 2, jj, pl.ds(k * L, L)]
                        macc = macc + va0 * (vb1 * va2)
                    mrow[pl.ds(j * L, L)] = macc
                    return 0

                lax.fori_loop(0, 16, p1_body, 0, unroll=False)
                mv = jnp.zeros((L,), jnp.float32)
                for k in range(L):
                    mv = mv + plsc.load_gather(mrow, [iot * L + k])
                mb[pl.ds(s * B1 + jb, L)] = mv
                xv = valb[pl.ds(bb * B1 + jb, L)]
                dv = 1.0 / (1.0 + jnp.exp(-mv)) - xv
                db[...] = dv

                def p2_body(j, _):
                    jj = jb + j
                    dvb = plsc.load_gather(db, [jnp.full((L,), j, jnp.int32)])
                    for k in range(8):
                        kd = pl.ds(k * L, L)
                        va0 = rows[s, 0, jj, kd]
                        vb1 = rows[s, 1, jj, kd]
                        va2 = rows[s, 2, jj, kd]
                        cob[s, 0, jj, kd] = dvb * (vb1 * va2)
                        cob[s, 1, jj, kd] = dvb * (va0 * va2)
                        cob[s, 2, jj, kd] = dvb * (va0 * vb1)
                    return 0

                lax.fori_loop(0, 16, p2_body, 0, unroll=False)
                return 0

            lax.fori_loop(0, 4, q_body, 0, unroll=False)

            blk = gbase + bb * B1
            wd = []
            for t in range(3):
                wd.append(pltpu.async_copy(
                    cob.at[s, t], co_out.at[pl.ds(t * NNZ_PAD + blk, B1)],
                    wsem))
            wd.append(pltpu.async_copy(
                mb.at[pl.ds(s * B1, B1)], m_out.at[pl.ds(blk, B1)], wsem))
            wdesc[bb] = wd

        for bb in sorted(wdesc):
            for d in wdesc[bb]:
                d.wait()
        return 0

    lax.fori_loop(0, NGRP, group, 0, unroll=False)


def _sc1(colsg, valsp, a, b):
    f = pl.kernel(
        _sc1_body,
        out_type=(
            jax.ShapeDtypeStruct((NNZ_PAD,), jnp.float32),
            jax.ShapeDtypeStruct((3 * NNZ_PAD, RANK), jnp.float32),
        ),
        mesh=_mesh(),
        compiler_params=_SC_PARAMS,
        scratch_types=[
            pltpu.VMEM((3 * B1 * GB,), jnp.int32),
            pltpu.VMEM((B1 * GB,), jnp.float32),
            pltpu.VMEM((2, 3, B1, RANK), jnp.float32),
            pltpu.VMEM((2, 3, B1, RANK), jnp.float32),
            pltpu.VMEM((2 * B1,), jnp.float32),
            pltpu.VMEM((L,), jnp.float32),
            pltpu.VMEM((L * L,), jnp.float32),
            pltpu.SemaphoreType.DMA,
            pltpu.SemaphoreType.DMA,
        ],
    )
    return f(colsg, valsp, a, b)


def _tcl_body(m_ref, x_ref, loss_ref):
    m = m_ref[...]
    x = x_ref[...]
    loss_ref[...] = jnp.logaddexp(0.0, m) - x * m


def _tc_loss(m_pad, x_pad):
    m2 = m_pad.reshape(NNZ_PAD // RANK, RANK)
    x2 = x_pad.reshape(NNZ_PAD // RANK, RANK)
    loss2 = pl.pallas_call(
        _tcl_body,
        out_shape=jax.ShapeDtypeStruct(m2.shape, jnp.float32),
    )(m2, x2)
    return loss2.reshape(-1)[:NNZ]


def _w1_body(elems3, w1_out, tbl, ebuf, esb):
    wid = _wid()
    wbase = wid * CHUNK1
    iot = _iota()

    for arr in range(3):
        def init(v, _):
            tbl[pl.ds(v * L, L)] = jnp.full((L,), -1, jnp.int32)
            return 0

        lax.fori_loop(0, TBL // L, init, 0, unroll=False)

        def ck_body(ck, _):
            base = wbase + ck * 2000
            pltpu.sync_copy(elems3.at[pl.ds(arr * NNZ_PAD + base, 2000)], ebuf)

            def v_body(v, _):
                e = ebuf[pl.ds(v * L, L)]
                key = (e << 4) | iot
                val = (base + v * L) + iot
                ks, vs = plsc.sort_key_val(key, val)
                es = ks >> 4
                esb[pl.ds(0, L)] = es
                esb[pl.ds(L, L)] = jnp.full((L,), 1 << 30, jnp.int32)
                nxt = esb[pl.ds(1, L)]
                keep = es != nxt
                plsc.store_scatter(tbl, [es], vs, mask=keep)
                return 0

            lax.fori_loop(0, 2000 // L, v_body, 0, unroll=False)
            return 0

        lax.fori_loop(0, CHUNK1 // 2000, ck_body, 0, unroll=False)
        pltpu.sync_copy(tbl, w1_out.at[pl.ds((arr * NW + wid) * TBL, TBL)])


def _w1(elems3):
    f = pl.kernel(
        _w1_body,
        out_type=jax.ShapeDtypeStruct((3 * NW * TBL,), jnp.int32),
        mesh=_mesh(),
        compiler_params=_SC_PARAMS,
        scratch_types=[
            pltpu.VMEM((TBL,), jnp.int32),
            pltpu.VMEM((2000,), jnp.int32),
            pltpu.VMEM((2 * L,), jnp.int32),
        ],
    )
    return f(elems3)


def _sc2_body(colsd, contrib, acc_out,
              chunk, zb, destb, plist, ibuf, drow, grows, zsem, gsem, ssem):
    core = lax.axis_index("c")
    sid = lax.axis_index("s")
    iot = _iota()

    def z_row(v, _):
        def z_col(k, _):
            zb[v, pl.ds(k * L, L)] = jnp.zeros((L,), jnp.float32)
            return 0
        lax.fori_loop(0, RANK // L, z_col, 0, unroll=False)
        return 0

    lax.fori_loop(0, L, z_row, 0, unroll=False)

    def pair_body(it, _):
        pair = it * NC + core
        m = pair // NP2
        p = pair % NP2
        rowbase = p * R2
        sbase = m * NNZ_PAD + sid * SCAN2

        zbase = sid * R2T
        zd = []
        for z in range(R2T // L):
            zd.append(pltpu.async_copy(
                zb, chunk.at[pl.ds(zbase + z * L, L)], zsem))
        zd.append(pltpu.async_copy(
            zb.at[pl.ds(0, R2T - (R2T // L) * L)],
            chunk.at[pl.ds(zbase + (R2T // L) * L, R2T - (R2T // L) * L)],
            zsem))

        def ck_body(ck, nh):
            pltpu.sync_copy(colsd.at[pl.ds(sbase + ck * 2000, 2000)], destb)

            def v_body(v, nh):
                dv = destb[pl.ds(v * L, L)]
                rel = dv - rowbase
                msk = (rel >= 0) & (rel < R2)
                off = (ck * 2000 + v * L) + iot
                plsc.store_compressed(plist.at[pl.ds(nh, L)],
                                      (off << RBITS) | rel, mask=msk)
                cnt = plsc.all_reduce_population_count(msk)
                return nh + cnt[0]

            return lax.fori_loop(0, 2000 // L, v_body, nh, unroll=False)

        nhits = lax.fori_loop(0, SCAN2 // 2000, ck_body, 0, unroll=False)

        def pad_body(k, _):
            plist[pl.ds(nhits + k * L, L)] = jnp.full((L,), R2 + sid,
                                                      jnp.int32)
            return 0

        lax.fori_loop(0, (KSB * 128) // L, pad_body, 0, unroll=False)
        nsuper = (nhits + KSB * 128 - 1) // (KSB * 128)

        for d in zd:
            d.wait()
        plsc.subcore_barrier()

        def sb_body(sbi, _):
            base = sbi * (KSB * 128)
            gd = []
            for s in range(KSB):
                def up(k, _):
                    pv = plist[pl.ds(base + s * 128 + k * L, L)]
                    ibuf[s, pl.ds(k * L, L)] = sbase + (pv >> RBITS)
                    drow[s, pl.ds(k * L, L)] = pv & ((1 << RBITS) - 1)
                    return 0
                lax.fori_loop(0, 128 // L, up, 0, unroll=False)
                gd.append(pltpu.async_copy(
                    contrib.at[ibuf.at[s]], grows.at[s], gsem))
            sd = []
            for s in range(KSB):
                gd[s].wait()
                sd.append(pltpu.async_copy(
                    grows.at[s], chunk.at[drow.at[s]], ssem, add=True))
            for d in sd:
                d.wait()
            return 0

        lax.fori_loop(0, nsuper, sb_body, 0, unroll=False)
        plsc.subcore_barrier()

        pltpu.sync_copy(
            chunk.at[pl.ds(sid * R2W, R2W)],
            acc_out.at[pl.ds(m * ACCM + p * R2 + sid * R2W, R2W)])
        plsc.subcore_barrier()
        return 0

    lax.fori_loop(0, (3 * NP2) // NC, pair_body, 0, unroll=False)

    @pl.when((core == 0) & (sid == 0))
    def _():
        pltpu.sync_copy(zb.at[pl.ds(0, L)], acc_out.at[pl.ds(ZROW, L)])


def _sc2(colsd, contrib):
    f = pl.kernel(
        _sc2_body,
        out_type=jax.ShapeDtypeStruct((ACC_ROWS, RANK), jnp.float32),
        mesh=_mesh(),
        compiler_params=_SC_PARAMS,
        scratch_types=[
            pltpu.VMEM_SHARED((R2 + L, RANK), jnp.float32),
            pltpu.VMEM((L, RANK), jnp.float32),
            pltpu.VMEM((2000,), jnp.int32),
            pltpu.VMEM((HMAX,), jnp.int32),
            pltpu.VMEM((KSB, 128), jnp.int32),
            pltpu.VMEM((KSB, 128), jnp.int32),
            pltpu.VMEM((KSB, 128, RANK), jnp.float32),
            pltpu.SemaphoreType.DMA,
            pltpu.SemaphoreType.DMA,
            pltpu.SemaphoreType.DMA,
        ],
    )
    return f(colsd, contrib)


def _sc3_body(w1, colsg, acc, a_out, b_out,
              winb, seg, cvb, srca, srcb, widx, grows):
    wid = _wid()
    ebase = wid * SEG

    for arr in range(3):
        def init(v, _):
            winb[pl.ds(arr * SEG + v * L, L)] = jnp.full((L,), -1, jnp.int32)
            return 0
        lax.fori_loop(0, SEG // L, init, 0, unroll=False)

        def t_body(t, _):
            pltpu.sync_copy(w1.at[pl.ds((arr * NW + t) * TBL + ebase, SEG)], seg)

            def v_body(v, _):
                wd = pl.ds(arr * SEG + v * L, L)
                vd = pl.ds(v * L, L)
                winb[wd] = jnp.maximum(winb[wd], seg[vd])
                return 0
            lax.fori_loop(0, SEG // L, v_body, 0, unroll=False)
            return 0
        lax.fori_loop(0, NW, t_body, 0, unroll=False)

    for arr in range(3):
        def c_body(v, _):
            wd = pl.ds(arr * SEG + v * L, L)
            widx[wd] = jnp.maximum(winb[wd], 0) + arr * NNZ_PAD
            return 0
        lax.fori_loop(0, SEG // L, c_body, 0, unroll=False)

        def g_body(bi, _):
            pltpu.sync_copy(
                colsg.at[widx.at[pl.ds(arr * SEG + bi * 128, 128)]],
                cvb.at[pl.ds(arr * SEG + bi * 128, 128)])
            return 0
        lax.fori_loop(0, SEG // 128, g_body, 0, unroll=False)

    def s_body(v, _):
        vd = pl.ds(v * L, L)
        wa = winb[pl.ds(0 * SEG + v * L, L)]
        wb = winb[pl.ds(1 * SEG + v * L, L)]
        wc = winb[pl.ds(2 * SEG + v * L, L)]
        cv0 = cvb[pl.ds(0 * SEG + v * L, L)]
        cv1 = cvb[pl.ds(1 * SEG + v * L, L)]
        cv2 = cvb[pl.ds(2 * SEG + v * L, L)]
        zr = jnp.full((L,), ZROW, jnp.int32)
        sa = jnp.where(wa >= 0, cv0, zr)
        sa = jnp.where(wc >= 0, 2 * ACCM + cv2, sa)
        sb = jnp.where(wb >= 0, ACCM + cv1, zr)
        srca[vd] = sa
        srcb[vd] = sb
        return 0

    lax.fori_loop(0, SEG // L, s_body, 0, unroll=False)

    def f_body(bi, _):
        pltpu.sync_copy(acc.at[srca.at[pl.ds(bi * 128, 128)]], grows)
        pltpu.sync_copy(grows, a_out.at[pl.ds(ebase + bi * 128, 128)])
        pltpu.sync_copy(acc.at[srcb.at[pl.ds(bi * 128, 128)]], grows)
        pltpu.sync_copy(grows, b_out.at[pl.ds(ebase + bi * 128, 128)])
        return 0

    lax.fori_loop(0, SEG // 128, f_body, 0, unroll=False)


def _sc3(w1, colsg, acc):
    f = pl.kernel(
        _sc3_body,
        out_type=(
            jax.ShapeDtypeStruct((TBL, RANK), jnp.float32),
            jax.ShapeDtypeStruct((TBL, RANK), jnp.float32),
        ),
        mesh=_mesh(),
        compiler_params=_SC_PARAMS,
        scratch_types=[
            pltpu.VMEM((3 * SEG,), jnp.int32),
            pltpu.VMEM((SEG,), jnp.int32),
            pltpu.VMEM((3 * SEG,), jnp.int32),
            pltpu.VMEM((SEG,), jnp.int32),
            pltpu.VMEM((SEG,), jnp.int32),
            pltpu.VMEM((3 * SEG,), jnp.int32),
            pltpu.VMEM((128, RANK), jnp.float32),
        ],
    )
    return f(w1, colsg, acc)


def kernel(coo_ns, vals_ns, a_elems, b_elems, c_elems, a, b):
    pad = NNZ_PAD - NNZ
    c0 = coo_ns[:, 0]
    c1 = coo_ns[:, 1]
    c2 = coo_ns[:, 2]
    colsg = jnp.concatenate([
        jnp.pad(c0, (0, pad)), jnp.pad(c1, (0, pad)), jnp.pad(c2, (0, pad))])
    colsd = jnp.concatenate([
        jnp.pad(c0, (0, pad), constant_values=SENT_DEST),
        jnp.pad(c1, (0, pad), constant_values=SENT_DEST),
        jnp.pad(c2, (0, pad), constant_values=SENT_DEST)])
    elems3 = jnp.concatenate([
        jnp.pad(a_elems, (0, pad), constant_values=N_ENT),
        jnp.pad(b_elems, (0, pad), constant_values=N_ENT),
        jnp.pad(c_elems, (0, pad), constant_values=N_ENT)])
    valsp = jnp.pad(vals_ns, (0, pad))

    m_pad, contrib = _sc1(colsg, valsp, a, b)
    loss = _tc_loss(m_pad, valsp)
    w1 = _w1(elems3)
    acc = _sc2(colsd, contrib)
    a_grad_pad, b_grad_pad = _sc3(w1, colsg, acc)
    return (loss, a_grad_pad[:N_ENT], b_grad_pad[:N_ENT])
